# Initial kernel scaffold; baseline (speedup 1.0000x reference)
import jax
import jax.numpy as jnp
from jax.experimental import pallas as pl


def kernel(x, edge_index, edge_attr, node_emb_w, node_emb_b, edge_emb_w, edge_emb_b, conv_w1, conv_b1, conv_w2, conv_b2, emlp_w1, emlp_b1, emlp_w2, emlp_b2, bn_gamma, bn_beta, mlp_w1, mlp_b1, mlp_w2, mlp_b2, mlp_w3, mlp_b3):
    raise NotImplementedError("write your pallas kernel here")


# speedup vs baseline: 1.6728x; 1.6728x over previous
import functools

import jax
import jax.numpy as jnp
from jax import lax
from jax.experimental import pallas as pl
from jax.experimental.pallas import tpu as pltpu
from jax.experimental.pallas import tpu_sc as plsc

_N = 10000
_E = 320000
_H = 100
_HP = 128
_BN_EPS = 1e-5

_NC = 2
_NS = 16
_NW = _NC * _NS
_K = 128
_CHUNKS = _E // _K
_BASE_CH = _CHUNKS // _NW
_EXTRA = _CHUNKS % _NW
_ZR = 16
_NHALF = _N // 2
_NAGG = 5120
_KA = 64
_CHA = _E // _KA

_PREC = None


def _mm_body(x_ref, w_ref, b_ref, o_ref):
    o_ref[...] = (
        jnp.dot(x_ref[...], w_ref[...], precision=_PREC) + b_ref[...]
    )


def _mm(x, w, b, block):
    n, d = x.shape
    dout = w.shape[1]
    return pl.pallas_call(
        _mm_body,
        grid=(n // block,),
        in_specs=[
            pl.BlockSpec((block, d), lambda i: (i, 0)),
            pl.BlockSpec((d, dout), lambda i: (0, 0)),
            pl.BlockSpec((1, dout), lambda i: (0, 0)),
        ],
        out_specs=pl.BlockSpec((block, dout), lambda i: (i, 0)),
        out_shape=jax.ShapeDtypeStruct((n, dout), jnp.float32),
    )(x, w, b)


def _edge_body(g_ref, e_ref, w1_ref, w2_ref, b2_ref, o_ref):
    e = e_ref[...]
    t = jnp.maximum(g_ref[...] + jnp.dot(e, w1_ref[...], precision=_PREC), 0.0)
    o_ref[...] = e + (jnp.dot(t, w2_ref[...], precision=_PREC) + b2_ref[...]) * 0.5


def _edge_update(gsum, e, w1c, w2e, b2e, block=2000):
    full = lambda a: pl.BlockSpec(a.shape, lambda i: (0,) * a.ndim)
    return pl.pallas_call(
        _edge_body,
        grid=(_E // block,),
        in_specs=[
            pl.BlockSpec((block, _HP), lambda i: (i, 0)),
            pl.BlockSpec((block, _HP), lambda i: (i, 0)),
            full(w1c), full(w2e), full(b2e),
        ],
        out_specs=pl.BlockSpec((block, _HP), lambda i: (i, 0)),
        out_shape=jax.ShapeDtypeStruct((_E, _HP), jnp.float32),
    )(gsum, e, w1c, w2e, b2e)


def _final_body(gu_ref, ga_ref, e_ref, w1c_ref, w2e_ref, b2e_ref, cw_ref,
                w2_ref, b2_ref, w3_ref, b3_ref, o_ref):
    e = e_ref[...]
    t = jnp.maximum(gu_ref[...] + jnp.dot(e, w1c_ref[...], precision=_PREC), 0.0)
    e2 = e + (jnp.dot(t, w2e_ref[...], precision=_PREC) + b2e_ref[...]) * 0.5
    q = jnp.maximum(ga_ref[...] + jnp.dot(e2, cw_ref[...], precision=_PREC), 0.0)
    q = jnp.maximum(jnp.dot(q, w2_ref[...], precision=_PREC) + b2_ref[...], 0.0)
    o_ref[...] = jnp.dot(q, w3_ref[...], precision=_PREC) + b3_ref[...]


def _final(gu, ga, e, w1c, w2e, b2e, cw, w2, b2, w3, b3, block=2000):
    full = lambda a: pl.BlockSpec(a.shape, lambda i: (0,) * a.ndim)
    return pl.pallas_call(
        _final_body,
        grid=(_E // block,),
        in_specs=[
            pl.BlockSpec((block, _HP), lambda i: (i, 0)),
            pl.BlockSpec((block, 64), lambda i: (i, 0)),
            pl.BlockSpec((block, _HP), lambda i: (i, 0)),
            full(w1c), full(w2e), full(b2e), full(cw),
            full(w2), full(b2), full(w3), full(b3),
        ],
        out_specs=pl.BlockSpec((block, 8), lambda i: (i, 0)),
        out_shape=jax.ShapeDtypeStruct((_E, 8), jnp.float32),
    )(gu, ga, e, w1c, w2e, b2e, cw, w2, b2, w3, b3)


def _node0_body(h_ref, parts_ref, w1_ref, b1_ref, w2_ref, b2_ref,
                g_ref, bt_ref, wa_ref, ba_ref, wb_ref,
                ho_ref, uo_ref, vo_ref):
    h = h_ref[...]
    p = parts_ref[...]
    z0 = h + jnp.concatenate(
        [p[:_NHALF], p[_NAGG:_NAGG + _NHALF]], axis=0)
    t = jnp.maximum(jnp.dot(z0, w1_ref[...], precision=_PREC) + b1_ref[...], 0.0)
    z = jnp.dot(t, w2_ref[...], precision=_PREC) + b2_ref[...]
    mu = jnp.mean(z, axis=0, keepdims=True)
    var = jnp.mean((z - mu) ** 2, axis=0, keepdims=True)
    zn = (z - mu) * (g_ref[...] * lax.rsqrt(var + _BN_EPS)) + bt_ref[...]
    hn = (h + jnp.maximum(zn, 0.0)) * 0.5
    ho_ref[...] = hn
    uo_ref[...] = jnp.dot(hn, wa_ref[...], precision=_PREC) + ba_ref[...]
    vo_ref[...] = jnp.dot(hn, wb_ref[...], precision=_PREC)


def _node0(h, parts, w1, b1, w2, b2, g, bt, wa, ba, wb):
    s = jax.ShapeDtypeStruct((_N, _HP), jnp.float32)
    return pl.pallas_call(
        _node0_body,
        out_shape=(s, s, s),
    )(h, parts, w1, b1, w2, b2, g, bt, wa, ba, wb)


def _node1_body(h_ref, parts_ref, w1_ref, b1_ref, w2_ref, b2_ref,
                g_ref, bt_ref, ho_ref):
    h = h_ref[...]
    p = parts_ref[...]
    z0 = h + jnp.concatenate(
        [p[:_NHALF], p[_NAGG:_NAGG + _NHALF]], axis=0)
    t = jnp.maximum(jnp.dot(z0, w1_ref[...], precision=_PREC) + b1_ref[...], 0.0)
    z = jnp.dot(t, w2_ref[...], precision=_PREC) + b2_ref[...]
    mu = jnp.mean(z, axis=0, keepdims=True)
    var = jnp.mean((z - mu) ** 2, axis=0, keepdims=True)
    zn = (z - mu) * (g_ref[...] * lax.rsqrt(var + _BN_EPS)) + bt_ref[...]
    ho_ref[...] = (h + jnp.maximum(zn, 0.0)) * 0.5


def _node1(h, parts, w1, b1, w2, b2, g, bt):
    s = jax.ShapeDtypeStruct((_N, _HP), jnp.float32)
    return pl.pallas_call(
        _node1_body,
        out_shape=s,
    )(h, parts, w1, b1, w2, b2, g, bt)


def _tables_body(h_ref, wa_ref, ba_ref, wb_ref, wa2_ref, ba2_ref, wb2_ref,
                 uo_ref, vo_ref, ao_ref, bo_ref):
    hn = h_ref[...]
    hr = jnp.maximum(hn, 0.0)
    uo_ref[...] = jnp.dot(hn, wa_ref[...], precision=_PREC) + ba_ref[...]
    vo_ref[...] = jnp.dot(hn, wb_ref[...], precision=_PREC)
    ao_ref[...] = jnp.dot(hr, wa2_ref[...], precision=_PREC) + ba2_ref[...]
    bo_ref[...] = jnp.dot(hr, wb2_ref[...], precision=_PREC)


def _tables(hn, wa, ba, wb, wa2, ba2, wb2, block=1000):
    full = lambda a: pl.BlockSpec(a.shape, lambda i: (0,) * a.ndim)
    s = jax.ShapeDtypeStruct((_N, _HP), jnp.float32)
    bs = pl.BlockSpec((block, _HP), lambda i: (i, 0))
    return pl.pallas_call(
        _tables_body,
        grid=(_N // block,),
        in_specs=[bs, full(wa), full(ba), full(wb),
                  full(wa2), full(ba2), full(wb2)],
        out_specs=(bs, bs, bs, bs),
        out_shape=(s, s, s, s),
    )(hn, wa, ba, wb, wa2, ba2, wb2)


@functools.cache
def _get_sc_agg():
  mesh = plsc.VectorSubcoreMesh(core_axis_name="c", subcore_axis_name="s")

  @functools.partial(
      pl.kernel,
      mesh=mesh,
      out_type=jax.ShapeDtypeStruct((2 * _NAGG, _HP), jnp.float32),
      scratch_types=[
          pltpu.VMEM((_KA,), jnp.int32),
          pltpu.VMEM((_KA,), jnp.int32),
          pltpu.VMEM((_KA, _HP), jnp.float32),
          pltpu.VMEM((_KA, _HP), jnp.float32),
          pltpu.VMEM((_KA, _HP), jnp.float32),
          pltpu.VMEM((_ZR, _HP), jnp.float32),
          pltpu.VMEM_SHARED((_NAGG, _HP), jnp.float32),
          pltpu.SemaphoreType.DMA,
      ],
  )
  def _sc_agg(h_hbm, e_hbm, src_hbm, dst_hbm, parts_hbm,
              idxs, idxd, rows, ebuf, msg, zbuf, agg, sem):
    c = lax.axis_index("c")
    s = lax.axis_index("s")

    def _zrow(r, carry):
        for v in range(_HP // 16):
            zbuf[r, v * 16:(v + 1) * 16] = jnp.zeros((16,), jnp.float32)
        return carry
    lax.fori_loop(0, _ZR, _zrow, 0)
    rbase = s * (_NAGG // _NS)
    for i in range(_NAGG // _NS // _ZR):
        pltpu.sync_copy(zbuf, agg.at[pl.ds(rbase + i * _ZR, _ZR)])
    plsc.subcore_barrier()

    nch = _CHA // _NS + jnp.where(s < _CHA % _NS, 1, 0)
    base = c * _NHALF

    def _chunk(jj, carry):
        off = (s + jj * _NS) * _KA
        pltpu.sync_copy(src_hbm.at[pl.ds(off, _KA)], idxs)
        pltpu.sync_copy(dst_hbm.at[pl.ds(off, _KA)], idxd)
        cp = pltpu.async_copy(h_hbm.at[idxs], rows, sem)
        pltpu.sync_copy(e_hbm.at[pl.ds(off, _KA)], ebuf)
        cp.wait()

        def _row(k, carry2):
            for v in range(_HP // 16):
                sl = slice(v * 16, (v + 1) * 16)
                msg[k, sl] = jnp.maximum(rows[k, sl] + ebuf[k, sl], 0.0)
            return carry2
        lax.fori_loop(0, _KA, _row, 0)

        def _fix(v, carry3):
            d = idxd[pl.ds(v * 16, 16)]
            t = d - base
            ok = (t >= 0) & (t < _NHALF)
            idxd[pl.ds(v * 16, 16)] = jnp.where(ok, t, _NAGG - 1)
            return carry3
        lax.fori_loop(0, _KA // 16, _fix, 0)
        pltpu.sync_copy(msg, agg.at[idxd], add=True)
        return carry
    lax.fori_loop(0, nch, _chunk, 0)

    plsc.subcore_barrier()

    def _wb(i, carry):
        r0 = rbase + i * _ZR
        pltpu.sync_copy(agg.at[pl.ds(r0, _ZR)], zbuf)
        pltpu.sync_copy(zbuf, parts_hbm.at[pl.ds(c * _NAGG + r0, _ZR)])
        return carry
    lax.fori_loop(0, _NAGG // _NS // _ZR, _wb, 0)

  return _sc_agg


@functools.cache
def _make_gsum(aux):
    mesh = plsc.VectorSubcoreMesh(core_axis_name="c", subcore_axis_name="s")
    outs = [jax.ShapeDtypeStruct((_E, _HP), jnp.float32)]
    scratch = [
        pltpu.VMEM((_K,), jnp.int32),
        pltpu.VMEM((_K,), jnp.int32),
        pltpu.VMEM((_K, _HP), jnp.float32),
        pltpu.VMEM((_K, _HP), jnp.float32),
        pltpu.VMEM((_K, _HP), jnp.float32),
    ]
    if aux:
        outs.append(jax.ShapeDtypeStruct((_E, 64), jnp.float32))
        scratch += [
            pltpu.VMEM((_K, _HP), jnp.float32),
            pltpu.VMEM((_K, _HP), jnp.float32),
            pltpu.VMEM((_K, 64), jnp.float32),
        ]
    scratch.append(pltpu.SemaphoreType.DMA)

    @functools.partial(
        pl.kernel,
        mesh=mesh,
        out_type=tuple(outs) if aux else outs[0],
        scratch_types=scratch,
    )
    def _gsum(*args):
        if aux:
            (u_hbm, v_hbm, ua_hbm, va_hbm, src_hbm, dst_hbm,
             gu_hbm, ga_hbm, idxs, idxd, ru, rv, sbu, rua, rva, sba,
             sem) = args
        else:
            (u_hbm, v_hbm, src_hbm, dst_hbm, gu_hbm,
             idxs, idxd, ru, rv, sbu, sem) = args
        c = lax.axis_index("c")
        s = lax.axis_index("s")
        wid = s * _NC + c
        nch = _BASE_CH + jnp.where(wid < _EXTRA, 1, 0)

        def _chunk(jj, carry):
            off = (wid + jj * _NW) * _K
            pltpu.sync_copy(src_hbm.at[pl.ds(off, _K)], idxs)
            pltpu.sync_copy(dst_hbm.at[pl.ds(off, _K)], idxd)
            cps = [pltpu.async_copy(u_hbm.at[idxs], ru, sem),
                   pltpu.async_copy(v_hbm.at[idxd], rv, sem)]
            if aux:
                cps += [pltpu.async_copy(ua_hbm.at[idxs], rua, sem),
                        pltpu.async_copy(va_hbm.at[idxd], rva, sem)]
            for cp in cps:
                cp.wait()

            def _row(k, carry2):
                for v in range(_HP // 16):
                    sl = slice(v * 16, (v + 1) * 16)
                    sbu[k, sl] = ru[k, sl] + rv[k, sl]
                if aux:
                    for v in range(64 // 16):
                        sl = slice(v * 16, (v + 1) * 16)
                        sba[k, sl] = rua[k, sl] + rva[k, sl]
                return carry2
            lax.fori_loop(0, _K, _row, 0)
            pltpu.sync_copy(sbu, gu_hbm.at[pl.ds(off, _K)])
            if aux:
                pltpu.sync_copy(sba, ga_hbm.at[pl.ds(off, _K)])
            return carry
        lax.fori_loop(0, nch, _chunk, 0)

    return _gsum


def kernel(x, edge_index, edge_attr, node_emb_w, node_emb_b, edge_emb_w,
           edge_emb_b, conv_w1, conv_b1, conv_w2, conv_b2, emlp_w1, emlp_b1,
           emlp_w2, emlp_b2, bn_gamma, bn_beta, mlp_w1, mlp_b1, mlp_w2,
           mlp_b2, mlp_w3, mlp_b3):
    H = _H
    src = edge_index[0]
    dst = edge_index[1]

    def pad_rc(w, r, c):
        return jnp.pad(w, ((0, r - w.shape[0]), (0, c - w.shape[1])))

    def pv(b, c):
        return jnp.pad(b, (0, c - b.shape[0]))[None, :]

    h0 = _mm(x, pad_rc(node_emb_w, _HP, _HP), pv(node_emb_b, _HP), 1000)
    e0 = _mm(edge_attr, pad_rc(edge_emb_w, 16, _HP), pv(edge_emb_b, _HP), 2000)

    sc_agg = _get_sc_agg()
    gsum1 = _make_gsum(False)
    gsum2 = _make_gsum(True)

    parts0 = sc_agg(h0, e0, src, dst)

    w1_0 = pad_rc(conv_w1[0], _HP, _HP)
    b1_0 = pv(conv_b1[0], _HP)
    w2_0 = pad_rc(conv_w2[0], _HP, _HP)
    b2_0 = pv(conv_b2[0], _HP)
    g_0 = pv(bn_gamma[0], _HP)
    bt_0 = pv(bn_beta[0], _HP)
    wa_0 = pad_rc(emlp_w1[0][:H], _HP, _HP)
    ba_0 = pv(emlp_b1[0], _HP)
    wb_0 = pad_rc(emlp_w1[0][H:2 * H], _HP, _HP)
    h1, u1, v1 = _node0(h0, parts0,
                        w1_0, b1_0, w2_0, b2_0, g_0, bt_0, wa_0, ba_0, wb_0)

    gs0 = gsum1(u1, v1, src, dst)
    w1c_0 = pad_rc(emlp_w1[0][2 * H:], _HP, _HP)
    w2e_0 = pad_rc(emlp_w2[0], _HP, _HP)
    b2e_0 = pv(emlp_b2[0], _HP)
    e1 = _edge_update(gs0, e0, w1c_0, w2e_0, b2e_0)

    parts1 = sc_agg(h1, e1, src, dst)

    w1_1 = pad_rc(conv_w1[1], _HP, _HP)
    b1_1 = pv(conv_b1[1], _HP)
    w2_1 = pad_rc(conv_w2[1], _HP, _HP)
    b2_1 = pv(conv_b2[1], _HP)
    g_1 = pv(bn_gamma[1], _HP)
    bt_1 = pv(bn_beta[1], _HP)
    wa_1 = pad_rc(emlp_w1[1][:H], _HP, _HP)
    ba_1 = pv(emlp_b1[1], _HP)
    wb_1 = pad_rc(emlp_w1[1][H:2 * H], _HP, _HP)
    wa_2 = pad_rc(mlp_w1[:H], _HP, _HP)
    ba_2 = pv(mlp_b1, _HP)
    wb_2 = pad_rc(mlp_w1[H:2 * H], _HP, _HP)
    h2 = _node1(h1, parts1, w1_1, b1_1, w2_1, b2_1, g_1, bt_1)
    u2, v2, a2, b2 = _tables(h2, wa_1, ba_1, wb_1, wa_2, ba_2, wb_2)

    gu, ga = gsum2(u2, v2, a2, b2, src, dst)

    w1c_1 = pad_rc(emlp_w1[1][2 * H:], _HP, _HP)
    w2e_1 = pad_rc(emlp_w2[1], _HP, _HP)
    b2e_1 = pv(emlp_b2[1], _HP)
    cw = pad_rc(mlp_w1[2 * H:], _HP, 64)
    w2p = pad_rc(mlp_w2, 64, 32)
    b2p = pv(mlp_b2, 32)
    w3p = pad_rc(mlp_w3, 32, 8)
    b3p = pv(mlp_b3, 8)
    out8 = _final(gu, ga, e1, w1c_1, w2e_1, b2e_1, cw, w2p, b2p, w3p, b3p)
    return out8[:, :2]
